# baseline (speedup 1.0000x reference)
import functools

import jax
import jax.numpy as jnp
from jax import lax
from jax.experimental import pallas as pl
from jax.experimental.pallas import tpu as pltpu
from jax.experimental.pallas import tpu_sc as plsc

N_NODES = 10000
N_PAD = 10240
E_EDGES = 320000
FEAT = 128
CHUNK = 400
AGG_CHUNK = 160
CLS_CHUNK = 200
BE = 1000

_f32 = jnp.float32
_mesh = plsc.VectorSubcoreMesh(core_axis_name="c", subcore_axis_name="s")


def _fill(ref, rows, width, value):
    vec = jnp.full((16,), value, _f32)
    if len(ref.shape) == 1:
        def body(i, _):
            ref[pl.ds(i * 16, 16)] = vec
            return 0
        lax.fori_loop(0, rows * width // 16, body, 0)
    else:
        def body(i, _):
            for k in range(width // 16):
                ref[i, pl.ds(k * 16, 16)] = vec
            return 0
        lax.fori_loop(0, rows, body, 0)


def _deg_body(dst_hbm, deg_out, idx_v, ones_v, zb_v, acc_sh, sem):
    c = lax.axis_index("c")
    s = lax.axis_index("s")
    _fill(zb_v, 640, 1, 0.0)
    _fill(ones_v, CHUNK, 1, 1.0)
    pltpu.sync_copy(zb_v, acc_sh.at[pl.ds(s * 640, 640)])
    plsc.subcore_barrier()
    per_tile = E_EDGES // 32
    base = (c * 16 + s) * per_tile

    def chunk(i, _):
        pltpu.sync_copy(dst_hbm.at[pl.ds(base + i * CHUNK, CHUNK)], idx_v)
        pltpu.sync_copy(ones_v, acc_sh.at[idx_v], add=True)
        return 0

    lax.fori_loop(0, per_tile // CHUNK, chunk, 0)
    plsc.subcore_barrier()
    pltpu.sync_copy(acc_sh.at[pl.ds(s * 640, 640)],
                    deg_out.at[pl.ds(c * N_PAD + s * 640, 640)])


_deg_call = functools.partial(
    pl.kernel, _deg_body, mesh=_mesh,
    out_type=jax.ShapeDtypeStruct((2 * N_PAD,), _f32),
    scratch_types=[
        pltpu.VMEM((CHUNK,), jnp.int32),
        pltpu.VMEM((CHUNK,), _f32),
        pltpu.VMEM((640,), _f32),
        pltpu.VMEM_SHARED((N_PAD,), _f32),
        pltpu.SemaphoreType.DMA,
    ],
)()


def _agg_body(g_hbm, src2_hbm, dst_hbm, out,
              sidx_a, didx_a, sidx_b, didx_b, rows_a, rows_b,
              acc_sh, sem_a, sem_b):
    c = lax.axis_index("c")
    s = lax.axis_index("s")
    _fill(rows_a, AGG_CHUNK, FEAT, 0.0)
    for j in range(4):
        pltpu.sync_copy(rows_a.at[pl.ds(0, AGG_CHUNK)],
                        acc_sh.at[pl.ds(s * 640 + j * AGG_CHUNK, AGG_CHUNK)])
    plsc.subcore_barrier()
    per_tile = E_EDGES // 16
    nch = per_tile // AGG_CHUNK
    base = c * E_EDGES + s * per_tile
    dbase = s * per_tile

    def load_idx(i, si, di):
        pltpu.sync_copy(src2_hbm.at[pl.ds(base + i * AGG_CHUNK, AGG_CHUNK)], si)
        pltpu.sync_copy(dst_hbm.at[pl.ds(dbase + i * AGG_CHUNK, AGG_CHUNK)], di)

    load_idx(0, sidx_a, didx_a)
    pltpu.async_copy(g_hbm.at[sidx_a], rows_a, sem_a).wait()
    pltpu.sync_copy(rows_a, acc_sh.at[didx_a], add=True)

    load_idx(1, sidx_a, didx_a)
    pltpu.async_copy(g_hbm.at[sidx_a], rows_a, sem_a)

    def ring(j, _):
        i0 = 1 + 2 * j
        i1 = i0 + 1
        load_idx(i1, sidx_b, didx_b)
        pltpu.make_async_copy(g_hbm.at[sidx_a], rows_a, sem_a).wait()
        pltpu.async_copy(g_hbm.at[sidx_b], rows_b, sem_b)
        pltpu.sync_copy(rows_a, acc_sh.at[didx_a], add=True)
        load_idx(lax.min(i1 + 1, nch - 1), sidx_a, didx_a)
        pltpu.make_async_copy(g_hbm.at[sidx_b], rows_b, sem_b).wait()
        pltpu.async_copy(g_hbm.at[sidx_a], rows_a, sem_a)
        pltpu.sync_copy(rows_b, acc_sh.at[didx_b], add=True)
        return 0

    lax.fori_loop(0, (nch - 1) // 2, ring, 0)
    pltpu.make_async_copy(g_hbm.at[sidx_a], rows_a, sem_a).wait()
    plsc.subcore_barrier()
    pltpu.sync_copy(acc_sh.at[pl.ds(s * 640, 640)],
                    out.at[pl.ds(c * N_PAD + s * 640, 640)])


_agg_call = functools.partial(
    pl.kernel, _agg_body, mesh=_mesh,
    out_type=jax.ShapeDtypeStruct((2 * N_PAD, FEAT), _f32),
    scratch_types=[
        pltpu.VMEM((AGG_CHUNK,), jnp.int32),
        pltpu.VMEM((AGG_CHUNK,), jnp.int32),
        pltpu.VMEM((AGG_CHUNK,), jnp.int32),
        pltpu.VMEM((AGG_CHUNK,), jnp.int32),
        pltpu.VMEM((AGG_CHUNK, FEAT), _f32),
        pltpu.VMEM((AGG_CHUNK, FEAT), _f32),
        pltpu.VMEM_SHARED((N_PAD, FEAT), _f32),
        pltpu.SemaphoreType.DMA,
        pltpu.SemaphoreType.DMA,
    ],
)()


def _clsg_body(a_hbm, b_hbm, src_hbm, dst_hbm, h0_out,
               sidx0, didx0, sidx1, didx1, ra0, rb0, ra1, rb1,
               sem_a0, sem_b0, sem_a1, sem_b1, *, e_half):
    c = lax.axis_index("c")
    s = lax.axis_index("s")
    per_tile = E_EDGES // 2 // 32
    base = e_half * (E_EDGES // 2) + (c * 16 + s) * per_tile
    obase = (c * 16 + s) * per_tile
    nch = per_tile // CLS_CHUNK

    def load_idx(i, si, di):
        pltpu.sync_copy(src_hbm.at[pl.ds(base + i * CLS_CHUNK, CLS_CHUNK)], si)
        pltpu.sync_copy(dst_hbm.at[pl.ds(base + i * CLS_CHUNK, CLS_CHUNK)], di)

    def addrows(ra, rb):
        def addrow(r, _):
            for k in range(FEAT // 16):
                sl = pl.ds(k * 16, 16)
                ra[r, sl] = ra[r, sl] + rb[r, sl]
            return 0
        lax.fori_loop(0, CLS_CHUNK, addrow, 0)

    load_idx(0, sidx0, didx0)
    pltpu.async_copy(a_hbm.at[sidx0], ra0, sem_a0)
    pltpu.async_copy(b_hbm.at[didx0], rb0, sem_b0)
    pltpu.make_async_copy(a_hbm.at[sidx0], ra0, sem_a0).wait()
    pltpu.make_async_copy(b_hbm.at[didx0], rb0, sem_b0).wait()
    addrows(ra0, rb0)
    pltpu.sync_copy(ra0, h0_out.at[pl.ds(obase, CLS_CHUNK)])

    load_idx(1, sidx0, didx0)
    pltpu.async_copy(a_hbm.at[sidx0], ra0, sem_a0)
    pltpu.async_copy(b_hbm.at[didx0], rb0, sem_b0)

    def ring(j, _):
        i0 = 1 + 2 * j
        i1 = i0 + 1
        load_idx(i1, sidx1, didx1)
        pltpu.async_copy(a_hbm.at[sidx1], ra1, sem_a1)
        pltpu.async_copy(b_hbm.at[didx1], rb1, sem_b1)
        pltpu.make_async_copy(a_hbm.at[sidx0], ra0, sem_a0).wait()
        pltpu.make_async_copy(b_hbm.at[didx0], rb0, sem_b0).wait()
        addrows(ra0, rb0)
        pltpu.sync_copy(ra0, h0_out.at[pl.ds(obase + i0 * CLS_CHUNK, CLS_CHUNK)])
        load_idx(lax.min(i1 + 1, nch - 1), sidx0, didx0)
        pltpu.async_copy(a_hbm.at[sidx0], ra0, sem_a0)
        pltpu.async_copy(b_hbm.at[didx0], rb0, sem_b0)
        pltpu.make_async_copy(a_hbm.at[sidx1], ra1, sem_a1).wait()
        pltpu.make_async_copy(b_hbm.at[didx1], rb1, sem_b1).wait()
        addrows(ra1, rb1)
        pltpu.sync_copy(ra1, h0_out.at[pl.ds(obase + i1 * CLS_CHUNK, CLS_CHUNK)])
        return 0

    lax.fori_loop(0, (nch - 1) // 2, ring, 0)
    pltpu.make_async_copy(a_hbm.at[sidx0], ra0, sem_a0).wait()
    pltpu.make_async_copy(b_hbm.at[didx0], rb0, sem_b0).wait()


_clsg_calls = [functools.partial(
    pl.kernel, functools.partial(_clsg_body, e_half=h), mesh=_mesh,
    out_type=jax.ShapeDtypeStruct((E_EDGES // 2, FEAT), _f32),
    scratch_types=[
        pltpu.VMEM((CLS_CHUNK,), jnp.int32),
        pltpu.VMEM((CLS_CHUNK,), jnp.int32),
        pltpu.VMEM((CLS_CHUNK,), jnp.int32),
        pltpu.VMEM((CLS_CHUNK,), jnp.int32),
        pltpu.VMEM((CLS_CHUNK, FEAT), _f32),
        pltpu.VMEM((CLS_CHUNK, FEAT), _f32),
        pltpu.VMEM((CLS_CHUNK, FEAT), _f32),
        pltpu.VMEM((CLS_CHUNK, FEAT), _f32),
        pltpu.SemaphoreType.DMA,
        pltpu.SemaphoreType.DMA,
        pltpu.SemaphoreType.DMA,
        pltpu.SemaphoreType.DMA,
    ],
)() for h in (0, 1)]


def _dot(a, b):
    return lax.dot_general(a, b, (((1,), (0,)), ((), ())),
                           preferred_element_type=_f32)


def _prep_tc(d0_ref, d1_ref, x_ref, wa_ref, wb_ref, dinv_o, g_o):
    deg = d0_ref[...] + d1_ref[...] + 1.0
    dinv = lax.rsqrt(deg)
    dinv_o[...] = dinv
    xv = x_ref[...]
    g_o[...] = jnp.concatenate(
        [_dot(xv, wa_ref[...]) * dinv, _dot(xv, wb_ref[...]) * dinv], axis=0)


def _norm_relu(acc, g, dinv, bias, bng, bnb):
    z = dinv * (acc + g) + bias
    mu = jnp.mean(z, axis=0, keepdims=True)
    var = jnp.mean((z - mu) ** 2, axis=0, keepdims=True)
    return jnp.maximum(bng * (z - mu) / jnp.sqrt(var + 1e-5) + bnb, 0.0)


def _branch_pair(accs_ref, g_ref, dinv_ref, ba_ref, bb_ref, bng_ref, bnb_ref):
    dinv = dinv_ref[...]
    accs = accs_ref[...]
    gv = g_ref[...]
    ha = _norm_relu(accs[:N_NODES], gv[:N_NODES], dinv, ba_ref[...],
                    bng_ref[...], bnb_ref[...])
    hb = _norm_relu(accs[N_PAD:N_PAD + N_NODES], gv[N_NODES:], dinv,
                    bb_ref[...], bng_ref[...], bnb_ref[...])
    return ha, hb, dinv


def _mid_tc(accs_ref, g_ref, dinv_ref,
            ba_ref, bb_ref, bng_ref, bnb_ref, wna_ref, wnb_ref, g_o):
    ha, hb, dinv = _branch_pair(accs_ref, g_ref, dinv_ref,
                                ba_ref, bb_ref, bng_ref, bnb_ref)
    g_o[...] = jnp.concatenate(
        [_dot(ha, wna_ref[...]) * dinv, _dot(hb, wnb_ref[...]) * dinv], axis=0)


def _fin_tc(accs_ref, g_ref, dinv_ref,
            ba_ref, bb_ref, bng_ref, bnb_ref, w1s_ref, w1d_ref,
            a_o, b_o):
    ha, hb, dinv = _branch_pair(accs_ref, g_ref, dinv_ref,
                                ba_ref, bb_ref, bng_ref, bnb_ref)
    xc = ha + hb
    a_o[...] = _dot(xc, w1s_ref[...])
    b_o[...] = _dot(xc, w1d_ref[...])


def _dotb(a, b):
    return lax.dot_general(a.astype(jnp.bfloat16), b.astype(jnp.bfloat16),
                           (((1,), (0,)), ((), ())),
                           preferred_element_type=_f32)


def _cls_tc(h0_ref, ea_ref, w1e_ref, b1_ref, w2_ref, b2_ref,
            w3_ref, b3_ref, w4_ref, b4_ref, w5_ref, b5_ref, out_ref):
    h = jnp.maximum(h0_ref[...] + _dot(ea_ref[...], w1e_ref[...])
                    + b1_ref[...], 0.0)
    h = jnp.maximum(_dotb(h, w2_ref[...]) + b2_ref[...], 0.0)
    h = jnp.maximum(_dotb(h, w3_ref[...]) + b3_ref[...], 0.0)
    h = jnp.maximum(_dotb(h, w4_ref[...]) + b4_ref[...], 0.0)
    out_ref[...] = _dotb(h, w5_ref[...]) + b5_ref[...]


def kernel(x, edge_index, edge_attr, params):
    p = params
    src = edge_index[0]
    dst = edge_index[1]

    deg_parts = _deg_call(dst)
    d0 = deg_parts[:N_NODES].reshape(N_NODES, 1)
    d1 = deg_parts[N_PAD:N_PAD + N_NODES].reshape(N_NODES, 1)

    dinv, g = pl.pallas_call(
        _prep_tc,
        out_shape=(jax.ShapeDtypeStruct((N_NODES, 1), _f32),
                   jax.ShapeDtypeStruct((2 * N_NODES, FEAT), _f32)),
    )(d0, d1, x, p['gcn1a_W'], p['gcn1b_W'])

    src2 = jnp.concatenate([src, src + jnp.int32(N_NODES)])

    def mid_layer(g, ba, bb, bng, bnb, wna, wnb):
        accs = _agg_call(g, src2, dst)
        return pl.pallas_call(
            _mid_tc,
            out_shape=jax.ShapeDtypeStruct((2 * N_NODES, FEAT), _f32),
        )(accs, g, dinv,
          ba.reshape(1, -1), bb.reshape(1, -1),
          bng.reshape(1, -1), bnb.reshape(1, -1), wna, wnb)

    g = mid_layer(g, p['gcn1a_b'], p['gcn1b_b'],
                  p['bn1_g'], p['bn1_b'], p['gcn2a_W'], p['gcn2b_W'])
    g = mid_layer(g, p['gcn2a_b'], p['gcn2b_b'],
                  p['bn2_g'], p['bn2_b'], p['gcn3a_W'], p['gcn3b_W'])

    accs = _agg_call(g, src2, dst)
    a_t, b_t = pl.pallas_call(
        _fin_tc,
        out_shape=(jax.ShapeDtypeStruct((N_NODES, FEAT), _f32),
                   jax.ShapeDtypeStruct((N_NODES, FEAT), _f32)),
    )(accs, g, dinv,
      p['gcn3a_b'].reshape(1, -1), p['gcn3b_b'].reshape(1, -1),
      p['bn3_g'].reshape(1, -1), p['bn3_b'].reshape(1, -1),
      p['cls_W1'][:FEAT], p['cls_W1'][FEAT:2 * FEAT])

    E2 = E_EDGES // 2
    nblk = E2 // BE

    def cls_half(h0, ea):
        return pl.pallas_call(
            _cls_tc,
            grid=(nblk,),
            in_specs=[
                pl.BlockSpec((BE, FEAT), lambda i: (i, 0)),
                pl.BlockSpec((BE, 16), lambda i: (i, 0)),
                pl.BlockSpec((16, FEAT), lambda i: (0, 0)),
                pl.BlockSpec((1, FEAT), lambda i: (0, 0)),
                pl.BlockSpec((FEAT, FEAT), lambda i: (0, 0)),
                pl.BlockSpec((1, FEAT), lambda i: (0, 0)),
                pl.BlockSpec((FEAT, 64), lambda i: (0, 0)),
                pl.BlockSpec((1, 64), lambda i: (0, 0)),
                pl.BlockSpec((64, 32), lambda i: (0, 0)),
                pl.BlockSpec((1, 32), lambda i: (0, 0)),
                pl.BlockSpec((32, 2), lambda i: (0, 0)),
                pl.BlockSpec((1, 2), lambda i: (0, 0)),
            ],
            out_specs=pl.BlockSpec((BE, 2), lambda i: (i, 0)),
            out_shape=jax.ShapeDtypeStruct((E2, 2), _f32),
        )(h0, ea, p['cls_W1'][2 * FEAT:], p['cls_b1'].reshape(1, -1),
          p['cls_W2'], p['cls_b2'].reshape(1, -1),
          p['cls_W3'], p['cls_b3'].reshape(1, -1),
          p['cls_W4'], p['cls_b4'].reshape(1, -1),
          p['cls_W5'], p['cls_b5'].reshape(1, -1))

    h0_0 = _clsg_calls[0](a_t, b_t, src, dst)
    h0_1 = _clsg_calls[1](a_t, b_t, src, dst)
    out0 = cls_half(h0_0, edge_attr[:E2])
    out1 = cls_half(h0_1, edge_attr[E2:])
    return jnp.concatenate([out0, out1], axis=0)


# speedup vs baseline: 1.0737x; 1.0737x over previous
import functools

import jax
import jax.numpy as jnp
from jax import lax
from jax.experimental import pallas as pl
from jax.experimental.pallas import tpu as pltpu
from jax.experimental.pallas import tpu_sc as plsc

N_NODES = 10000
N_PAD = 10240
E_EDGES = 320000
FEAT = 128
CHUNK = 400
AGG_CHUNK = 160
CLS_CHUNK = 200
BE = 2000

_f32 = jnp.float32
_mesh = plsc.VectorSubcoreMesh(core_axis_name="c", subcore_axis_name="s")


def _fill(ref, rows, width, value):
    vec = jnp.full((16,), value, _f32)
    if len(ref.shape) == 1:
        def body(i, _):
            ref[pl.ds(i * 16, 16)] = vec
            return 0
        lax.fori_loop(0, rows * width // 16, body, 0)
    else:
        def body(i, _):
            for k in range(width // 16):
                ref[i, pl.ds(k * 16, 16)] = vec
            return 0
        lax.fori_loop(0, rows, body, 0)


def _deg_body(dst_hbm, deg_out, idx_v, ones_v, zb_v, acc_sh, sem):
    c = lax.axis_index("c")
    s = lax.axis_index("s")
    _fill(zb_v, 640, 1, 0.0)
    _fill(ones_v, CHUNK, 1, 1.0)
    pltpu.sync_copy(zb_v, acc_sh.at[pl.ds(s * 640, 640)])
    plsc.subcore_barrier()
    per_tile = E_EDGES // 32
    base = (c * 16 + s) * per_tile

    def chunk(i, _):
        pltpu.sync_copy(dst_hbm.at[pl.ds(base + i * CHUNK, CHUNK)], idx_v)
        pltpu.sync_copy(ones_v, acc_sh.at[idx_v], add=True)
        return 0

    lax.fori_loop(0, per_tile // CHUNK, chunk, 0)
    plsc.subcore_barrier()
    pltpu.sync_copy(acc_sh.at[pl.ds(s * 640, 640)],
                    deg_out.at[pl.ds(c * N_PAD + s * 640, 640)])


_deg_call = functools.partial(
    pl.kernel, _deg_body, mesh=_mesh,
    out_type=jax.ShapeDtypeStruct((2 * N_PAD,), _f32),
    scratch_types=[
        pltpu.VMEM((CHUNK,), jnp.int32),
        pltpu.VMEM((CHUNK,), _f32),
        pltpu.VMEM((640,), _f32),
        pltpu.VMEM_SHARED((N_PAD,), _f32),
        pltpu.SemaphoreType.DMA,
    ],
)()


def _agg_body(g_hbm, src2_hbm, dst_hbm, out,
              sidx_a, didx_a, sidx_b, didx_b, rows_a, rows_b,
              acc_sh, sem_a, sem_b):
    c = lax.axis_index("c")
    s = lax.axis_index("s")
    _fill(rows_a, AGG_CHUNK, FEAT, 0.0)
    for j in range(4):
        pltpu.sync_copy(rows_a.at[pl.ds(0, AGG_CHUNK)],
                        acc_sh.at[pl.ds(s * 640 + j * AGG_CHUNK, AGG_CHUNK)])
    plsc.subcore_barrier()
    per_tile = E_EDGES // 16
    nch = per_tile // AGG_CHUNK
    base = c * E_EDGES + s * per_tile
    dbase = s * per_tile

    def load_idx(i, si, di):
        pltpu.sync_copy(src2_hbm.at[pl.ds(base + i * AGG_CHUNK, AGG_CHUNK)], si)
        pltpu.sync_copy(dst_hbm.at[pl.ds(dbase + i * AGG_CHUNK, AGG_CHUNK)], di)

    load_idx(0, sidx_a, didx_a)
    pltpu.async_copy(g_hbm.at[sidx_a], rows_a, sem_a).wait()
    pltpu.sync_copy(rows_a, acc_sh.at[didx_a], add=True)

    load_idx(1, sidx_a, didx_a)
    pltpu.async_copy(g_hbm.at[sidx_a], rows_a, sem_a)

    def ring(j, _):
        i0 = 1 + 2 * j
        i1 = i0 + 1
        load_idx(i1, sidx_b, didx_b)
        pltpu.make_async_copy(g_hbm.at[sidx_a], rows_a, sem_a).wait()
        pltpu.async_copy(g_hbm.at[sidx_b], rows_b, sem_b)
        pltpu.sync_copy(rows_a, acc_sh.at[didx_a], add=True)
        load_idx(lax.min(i1 + 1, nch - 1), sidx_a, didx_a)
        pltpu.make_async_copy(g_hbm.at[sidx_b], rows_b, sem_b).wait()
        pltpu.async_copy(g_hbm.at[sidx_a], rows_a, sem_a)
        pltpu.sync_copy(rows_b, acc_sh.at[didx_b], add=True)
        return 0

    lax.fori_loop(0, (nch - 1) // 2, ring, 0)
    pltpu.make_async_copy(g_hbm.at[sidx_a], rows_a, sem_a).wait()
    plsc.subcore_barrier()
    pltpu.sync_copy(acc_sh.at[pl.ds(s * 640, 640)],
                    out.at[pl.ds(c * N_PAD + s * 640, 640)])


_agg_call = functools.partial(
    pl.kernel, _agg_body, mesh=_mesh,
    out_type=jax.ShapeDtypeStruct((2 * N_PAD, FEAT), _f32),
    scratch_types=[
        pltpu.VMEM((AGG_CHUNK,), jnp.int32),
        pltpu.VMEM((AGG_CHUNK,), jnp.int32),
        pltpu.VMEM((AGG_CHUNK,), jnp.int32),
        pltpu.VMEM((AGG_CHUNK,), jnp.int32),
        pltpu.VMEM((AGG_CHUNK, FEAT), _f32),
        pltpu.VMEM((AGG_CHUNK, FEAT), _f32),
        pltpu.VMEM_SHARED((N_PAD, FEAT), _f32),
        pltpu.SemaphoreType.DMA,
        pltpu.SemaphoreType.DMA,
    ],
)()


def _clsg_body(a_hbm, b_hbm, src_hbm, dst_hbm, h0_out,
               sidx0, didx0, sidx1, didx1, ra0, rb0, ra1, rb1,
               sem_a0, sem_b0, sem_a1, sem_b1, *, e_half):
    c = lax.axis_index("c")
    s = lax.axis_index("s")
    per_tile = E_EDGES // 2 // 32
    base = e_half * (E_EDGES // 2) + (c * 16 + s) * per_tile
    obase = (c * 16 + s) * per_tile
    nch = per_tile // CLS_CHUNK

    def load_idx(i, si, di):
        pltpu.sync_copy(src_hbm.at[pl.ds(base + i * CLS_CHUNK, CLS_CHUNK)], si)
        pltpu.sync_copy(dst_hbm.at[pl.ds(base + i * CLS_CHUNK, CLS_CHUNK)], di)

    def addrows(ra, rb):
        def addrow(r, _):
            for k in range(FEAT // 16):
                sl = pl.ds(k * 16, 16)
                ra[r, sl] = ra[r, sl] + rb[r, sl]
            return 0
        lax.fori_loop(0, CLS_CHUNK, addrow, 0)

    load_idx(0, sidx0, didx0)
    pltpu.async_copy(a_hbm.at[sidx0], ra0, sem_a0)
    pltpu.async_copy(b_hbm.at[didx0], rb0, sem_b0)
    pltpu.make_async_copy(a_hbm.at[sidx0], ra0, sem_a0).wait()
    pltpu.make_async_copy(b_hbm.at[didx0], rb0, sem_b0).wait()
    addrows(ra0, rb0)
    pltpu.sync_copy(ra0, h0_out.at[pl.ds(obase, CLS_CHUNK)])

    load_idx(1, sidx0, didx0)
    pltpu.async_copy(a_hbm.at[sidx0], ra0, sem_a0)
    pltpu.async_copy(b_hbm.at[didx0], rb0, sem_b0)

    def ring(j, _):
        i0 = 1 + 2 * j
        i1 = i0 + 1
        load_idx(i1, sidx1, didx1)
        pltpu.async_copy(a_hbm.at[sidx1], ra1, sem_a1)
        pltpu.async_copy(b_hbm.at[didx1], rb1, sem_b1)
        pltpu.make_async_copy(a_hbm.at[sidx0], ra0, sem_a0).wait()
        pltpu.make_async_copy(b_hbm.at[didx0], rb0, sem_b0).wait()
        addrows(ra0, rb0)
        pltpu.sync_copy(ra0, h0_out.at[pl.ds(obase + i0 * CLS_CHUNK, CLS_CHUNK)])
        load_idx(lax.min(i1 + 1, nch - 1), sidx0, didx0)
        pltpu.async_copy(a_hbm.at[sidx0], ra0, sem_a0)
        pltpu.async_copy(b_hbm.at[didx0], rb0, sem_b0)
        pltpu.make_async_copy(a_hbm.at[sidx1], ra1, sem_a1).wait()
        pltpu.make_async_copy(b_hbm.at[didx1], rb1, sem_b1).wait()
        addrows(ra1, rb1)
        pltpu.sync_copy(ra1, h0_out.at[pl.ds(obase + i1 * CLS_CHUNK, CLS_CHUNK)])
        return 0

    lax.fori_loop(0, (nch - 1) // 2, ring, 0)
    pltpu.make_async_copy(a_hbm.at[sidx0], ra0, sem_a0).wait()
    pltpu.make_async_copy(b_hbm.at[didx0], rb0, sem_b0).wait()


_clsg_calls = [functools.partial(
    pl.kernel, functools.partial(_clsg_body, e_half=h), mesh=_mesh,
    out_type=jax.ShapeDtypeStruct((E_EDGES // 2, FEAT), _f32),
    scratch_types=[
        pltpu.VMEM((CLS_CHUNK,), jnp.int32),
        pltpu.VMEM((CLS_CHUNK,), jnp.int32),
        pltpu.VMEM((CLS_CHUNK,), jnp.int32),
        pltpu.VMEM((CLS_CHUNK,), jnp.int32),
        pltpu.VMEM((CLS_CHUNK, FEAT), _f32),
        pltpu.VMEM((CLS_CHUNK, FEAT), _f32),
        pltpu.VMEM((CLS_CHUNK, FEAT), _f32),
        pltpu.VMEM((CLS_CHUNK, FEAT), _f32),
        pltpu.SemaphoreType.DMA,
        pltpu.SemaphoreType.DMA,
        pltpu.SemaphoreType.DMA,
        pltpu.SemaphoreType.DMA,
    ],
)() for h in (0, 1)]


def _dot(a, b):
    return lax.dot_general(a, b, (((1,), (0,)), ((), ())),
                           preferred_element_type=_f32)


def _prep_tc(d0_ref, d1_ref, x_ref, wa_ref, wb_ref, dinv_o, g_o):
    deg = d0_ref[...] + d1_ref[...] + 1.0
    dinv = lax.rsqrt(deg)
    dinv_o[...] = dinv
    xv = x_ref[...]
    g_o[...] = jnp.concatenate(
        [_dot(xv, wa_ref[...]) * dinv, _dot(xv, wb_ref[...]) * dinv], axis=0)


def _norm_relu(acc, g, dinv, bias, bng, bnb):
    z = dinv * (acc + g) + bias
    mu = jnp.mean(z, axis=0, keepdims=True)
    var = jnp.mean((z - mu) ** 2, axis=0, keepdims=True)
    return jnp.maximum(bng * (z - mu) / jnp.sqrt(var + 1e-5) + bnb, 0.0)


def _branch_pair(accs_ref, g_ref, dinv_ref, ba_ref, bb_ref, bng_ref, bnb_ref):
    dinv = dinv_ref[...]
    accs = accs_ref[...]
    gv = g_ref[...]
    ha = _norm_relu(accs[:N_NODES], gv[:N_NODES], dinv, ba_ref[...],
                    bng_ref[...], bnb_ref[...])
    hb = _norm_relu(accs[N_PAD:N_PAD + N_NODES], gv[N_NODES:], dinv,
                    bb_ref[...], bng_ref[...], bnb_ref[...])
    return ha, hb, dinv


def _mid_tc(accs_ref, g_ref, dinv_ref,
            ba_ref, bb_ref, bng_ref, bnb_ref, wna_ref, wnb_ref, g_o):
    ha, hb, dinv = _branch_pair(accs_ref, g_ref, dinv_ref,
                                ba_ref, bb_ref, bng_ref, bnb_ref)
    g_o[...] = jnp.concatenate(
        [_dot(ha, wna_ref[...]) * dinv, _dot(hb, wnb_ref[...]) * dinv], axis=0)


def _fin_tc(accs_ref, g_ref, dinv_ref,
            ba_ref, bb_ref, bng_ref, bnb_ref, w1s_ref, w1d_ref,
            a_o, b_o):
    ha, hb, dinv = _branch_pair(accs_ref, g_ref, dinv_ref,
                                ba_ref, bb_ref, bng_ref, bnb_ref)
    xc = ha + hb
    a_o[...] = _dot(xc, w1s_ref[...])
    b_o[...] = _dot(xc, w1d_ref[...])


def _dotb(a, b):
    return lax.dot_general(a.astype(jnp.bfloat16), b.astype(jnp.bfloat16),
                           (((1,), (0,)), ((), ())),
                           preferred_element_type=_f32)


def _cls_tc(h0_ref, ea_ref, w1e_ref, b1_ref, w2_ref, b2_ref,
            w3_ref, b3_ref, w4_ref, b4_ref, w5_ref, b5_ref, out_ref):
    h = jnp.maximum(h0_ref[...] + _dot(ea_ref[...], w1e_ref[...])
                    + b1_ref[...], 0.0)
    h = jnp.maximum(_dotb(h, w2_ref[...]) + b2_ref[...], 0.0)
    h = jnp.maximum(_dotb(h, w3_ref[...]) + b3_ref[...], 0.0)
    h = jnp.maximum(_dotb(h, w4_ref[...]) + b4_ref[...], 0.0)
    out_ref[...] = _dotb(h, w5_ref[...]) + b5_ref[...]


def kernel(x, edge_index, edge_attr, params):
    p = params
    src = edge_index[0]
    dst = edge_index[1]

    deg_parts = _deg_call(dst)
    d0 = deg_parts[:N_NODES].reshape(N_NODES, 1)
    d1 = deg_parts[N_PAD:N_PAD + N_NODES].reshape(N_NODES, 1)

    dinv, g = pl.pallas_call(
        _prep_tc,
        out_shape=(jax.ShapeDtypeStruct((N_NODES, 1), _f32),
                   jax.ShapeDtypeStruct((2 * N_NODES, FEAT), _f32)),
    )(d0, d1, x, p['gcn1a_W'], p['gcn1b_W'])

    src2 = jnp.concatenate([src, src + jnp.int32(N_NODES)])

    def mid_layer(g, ba, bb, bng, bnb, wna, wnb):
        accs = _agg_call(g, src2, dst)
        return pl.pallas_call(
            _mid_tc,
            out_shape=jax.ShapeDtypeStruct((2 * N_NODES, FEAT), _f32),
        )(accs, g, dinv,
          ba.reshape(1, -1), bb.reshape(1, -1),
          bng.reshape(1, -1), bnb.reshape(1, -1), wna, wnb)

    g = mid_layer(g, p['gcn1a_b'], p['gcn1b_b'],
                  p['bn1_g'], p['bn1_b'], p['gcn2a_W'], p['gcn2b_W'])
    g = mid_layer(g, p['gcn2a_b'], p['gcn2b_b'],
                  p['bn2_g'], p['bn2_b'], p['gcn3a_W'], p['gcn3b_W'])

    accs = _agg_call(g, src2, dst)
    a_t, b_t = pl.pallas_call(
        _fin_tc,
        out_shape=(jax.ShapeDtypeStruct((N_NODES, FEAT), _f32),
                   jax.ShapeDtypeStruct((N_NODES, FEAT), _f32)),
    )(accs, g, dinv,
      p['gcn3a_b'].reshape(1, -1), p['gcn3b_b'].reshape(1, -1),
      p['bn3_g'].reshape(1, -1), p['bn3_b'].reshape(1, -1),
      p['cls_W1'][:FEAT], p['cls_W1'][FEAT:2 * FEAT])

    E2 = E_EDGES // 2
    nblk = E2 // BE

    def cls_half(h0, ea):
        return pl.pallas_call(
            _cls_tc,
            grid=(nblk,),
            in_specs=[
                pl.BlockSpec((BE, FEAT), lambda i: (i, 0)),
                pl.BlockSpec((BE, 16), lambda i: (i, 0)),
                pl.BlockSpec((16, FEAT), lambda i: (0, 0)),
                pl.BlockSpec((1, FEAT), lambda i: (0, 0)),
                pl.BlockSpec((FEAT, FEAT), lambda i: (0, 0)),
                pl.BlockSpec((1, FEAT), lambda i: (0, 0)),
                pl.BlockSpec((FEAT, 64), lambda i: (0, 0)),
                pl.BlockSpec((1, 64), lambda i: (0, 0)),
                pl.BlockSpec((64, 32), lambda i: (0, 0)),
                pl.BlockSpec((1, 32), lambda i: (0, 0)),
                pl.BlockSpec((32, 2), lambda i: (0, 0)),
                pl.BlockSpec((1, 2), lambda i: (0, 0)),
            ],
            out_specs=pl.BlockSpec((BE, 2), lambda i: (i, 0)),
            out_shape=jax.ShapeDtypeStruct((E2, 2), _f32),
        )(h0, ea, p['cls_W1'][2 * FEAT:], p['cls_b1'].reshape(1, -1),
          p['cls_W2'], p['cls_b2'].reshape(1, -1),
          p['cls_W3'], p['cls_b3'].reshape(1, -1),
          p['cls_W4'], p['cls_b4'].reshape(1, -1),
          p['cls_W5'], p['cls_b5'].reshape(1, -1))

    h0_0 = _clsg_calls[0](a_t, b_t, src, dst)
    h0_1 = _clsg_calls[1](a_t, b_t, src, dst)
    out0 = cls_half(h0_0, edge_attr[:E2])
    out1 = cls_half(h0_1, edge_attr[E2:])
    return jnp.concatenate([out0, out1], axis=0)


# speedup vs baseline: 1.1304x; 1.0527x over previous
import functools

import jax
import jax.numpy as jnp
from jax import lax
from jax.experimental import pallas as pl
from jax.experimental.pallas import tpu as pltpu
from jax.experimental.pallas import tpu_sc as plsc

N_NODES = 10000
N_PAD = 10240
E_EDGES = 320000
FEAT = 128
CHUNK = 400
AGG_CHUNK = 160
CLS_CHUNK = 200
BE = 4000

_f32 = jnp.float32
_mesh = plsc.VectorSubcoreMesh(core_axis_name="c", subcore_axis_name="s")


def _fill(ref, rows, width, value):
    vec = jnp.full((16,), value, _f32)
    if len(ref.shape) == 1:
        def body(i, _):
            ref[pl.ds(i * 16, 16)] = vec
            return 0
        lax.fori_loop(0, rows * width // 16, body, 0)
    else:
        def body(i, _):
            for k in range(width // 16):
                ref[i, pl.ds(k * 16, 16)] = vec
            return 0
        lax.fori_loop(0, rows, body, 0)


def _deg_body(dst_hbm, deg_out, idx_v, ones_v, zb_v, acc_sh, sem):
    c = lax.axis_index("c")
    s = lax.axis_index("s")
    _fill(zb_v, 640, 1, 0.0)
    _fill(ones_v, CHUNK, 1, 1.0)
    pltpu.sync_copy(zb_v, acc_sh.at[pl.ds(s * 640, 640)])
    plsc.subcore_barrier()
    per_tile = E_EDGES // 32
    base = (c * 16 + s) * per_tile

    def chunk(i, _):
        pltpu.sync_copy(dst_hbm.at[pl.ds(base + i * CHUNK, CHUNK)], idx_v)
        pltpu.sync_copy(ones_v, acc_sh.at[idx_v], add=True)
        return 0

    lax.fori_loop(0, per_tile // CHUNK, chunk, 0)
    plsc.subcore_barrier()
    pltpu.sync_copy(acc_sh.at[pl.ds(s * 640, 640)],
                    deg_out.at[pl.ds(c * N_PAD + s * 640, 640)])


_deg_call = functools.partial(
    pl.kernel, _deg_body, mesh=_mesh,
    out_type=jax.ShapeDtypeStruct((2 * N_PAD,), _f32),
    scratch_types=[
        pltpu.VMEM((CHUNK,), jnp.int32),
        pltpu.VMEM((CHUNK,), _f32),
        pltpu.VMEM((640,), _f32),
        pltpu.VMEM_SHARED((N_PAD,), _f32),
        pltpu.SemaphoreType.DMA,
    ],
)()


def _agg_body(g_hbm, src2_hbm, dst_hbm, out,
              sidx_a, didx_a, sidx_b, didx_b, rows_a, rows_b,
              acc_sh, sem_a, sem_b):
    c = lax.axis_index("c")
    s = lax.axis_index("s")
    _fill(rows_a, AGG_CHUNK, FEAT, 0.0)
    for j in range(4):
        pltpu.sync_copy(rows_a.at[pl.ds(0, AGG_CHUNK)],
                        acc_sh.at[pl.ds(s * 640 + j * AGG_CHUNK, AGG_CHUNK)])
    plsc.subcore_barrier()
    per_tile = E_EDGES // 16
    nch = per_tile // AGG_CHUNK
    base = c * E_EDGES + s * per_tile
    dbase = s * per_tile

    def load_idx(i, si, di):
        pltpu.sync_copy(src2_hbm.at[pl.ds(base + i * AGG_CHUNK, AGG_CHUNK)], si)
        pltpu.sync_copy(dst_hbm.at[pl.ds(dbase + i * AGG_CHUNK, AGG_CHUNK)], di)

    load_idx(0, sidx_a, didx_a)
    pltpu.async_copy(g_hbm.at[sidx_a], rows_a, sem_a).wait()
    pltpu.sync_copy(rows_a, acc_sh.at[didx_a], add=True)

    load_idx(1, sidx_a, didx_a)
    pltpu.async_copy(g_hbm.at[sidx_a], rows_a, sem_a)

    def ring(j, _):
        i0 = 1 + 2 * j
        i1 = i0 + 1
        load_idx(i1, sidx_b, didx_b)
        pltpu.make_async_copy(g_hbm.at[sidx_a], rows_a, sem_a).wait()
        pltpu.async_copy(g_hbm.at[sidx_b], rows_b, sem_b)
        pltpu.sync_copy(rows_a, acc_sh.at[didx_a], add=True)
        load_idx(lax.min(i1 + 1, nch - 1), sidx_a, didx_a)
        pltpu.make_async_copy(g_hbm.at[sidx_b], rows_b, sem_b).wait()
        pltpu.async_copy(g_hbm.at[sidx_a], rows_a, sem_a)
        pltpu.sync_copy(rows_b, acc_sh.at[didx_b], add=True)
        return 0

    lax.fori_loop(0, (nch - 1) // 2, ring, 0)
    pltpu.make_async_copy(g_hbm.at[sidx_a], rows_a, sem_a).wait()
    plsc.subcore_barrier()
    pltpu.sync_copy(acc_sh.at[pl.ds(s * 640, 640)],
                    out.at[pl.ds(c * N_PAD + s * 640, 640)])


_agg_call = functools.partial(
    pl.kernel, _agg_body, mesh=_mesh,
    out_type=jax.ShapeDtypeStruct((2 * N_PAD, FEAT), _f32),
    scratch_types=[
        pltpu.VMEM((AGG_CHUNK,), jnp.int32),
        pltpu.VMEM((AGG_CHUNK,), jnp.int32),
        pltpu.VMEM((AGG_CHUNK,), jnp.int32),
        pltpu.VMEM((AGG_CHUNK,), jnp.int32),
        pltpu.VMEM((AGG_CHUNK, FEAT), _f32),
        pltpu.VMEM((AGG_CHUNK, FEAT), _f32),
        pltpu.VMEM_SHARED((N_PAD, FEAT), _f32),
        pltpu.SemaphoreType.DMA,
        pltpu.SemaphoreType.DMA,
    ],
)()


def _clsg_body(a_hbm, b_hbm, src_hbm, dst_hbm, h0_out,
               sidx0, didx0, sidx1, didx1, ra0, rb0, ra1, rb1,
               sem_a0, sem_b0, sem_a1, sem_b1, *, e_half):
    c = lax.axis_index("c")
    s = lax.axis_index("s")
    per_tile = E_EDGES // 2 // 32
    base = e_half * (E_EDGES // 2) + (c * 16 + s) * per_tile
    obase = (c * 16 + s) * per_tile
    nch = per_tile // CLS_CHUNK

    def load_idx(i, si, di):
        pltpu.sync_copy(src_hbm.at[pl.ds(base + i * CLS_CHUNK, CLS_CHUNK)], si)
        pltpu.sync_copy(dst_hbm.at[pl.ds(base + i * CLS_CHUNK, CLS_CHUNK)], di)

    def addrows(ra, rb):
        def addrow(r, _):
            for k in range(FEAT // 16):
                sl = pl.ds(k * 16, 16)
                ra[r, sl] = ra[r, sl] + rb[r, sl]
            return 0
        lax.fori_loop(0, CLS_CHUNK, addrow, 0)

    load_idx(0, sidx0, didx0)
    pltpu.async_copy(a_hbm.at[sidx0], ra0, sem_a0)
    pltpu.async_copy(b_hbm.at[didx0], rb0, sem_b0)
    pltpu.make_async_copy(a_hbm.at[sidx0], ra0, sem_a0).wait()
    pltpu.make_async_copy(b_hbm.at[didx0], rb0, sem_b0).wait()
    addrows(ra0, rb0)
    pltpu.sync_copy(ra0, h0_out.at[pl.ds(obase, CLS_CHUNK)])

    load_idx(1, sidx0, didx0)
    pltpu.async_copy(a_hbm.at[sidx0], ra0, sem_a0)
    pltpu.async_copy(b_hbm.at[didx0], rb0, sem_b0)

    def ring(j, _):
        i0 = 1 + 2 * j
        i1 = i0 + 1
        load_idx(i1, sidx1, didx1)
        pltpu.async_copy(a_hbm.at[sidx1], ra1, sem_a1)
        pltpu.async_copy(b_hbm.at[didx1], rb1, sem_b1)
        pltpu.make_async_copy(a_hbm.at[sidx0], ra0, sem_a0).wait()
        pltpu.make_async_copy(b_hbm.at[didx0], rb0, sem_b0).wait()
        addrows(ra0, rb0)
        pltpu.sync_copy(ra0, h0_out.at[pl.ds(obase + i0 * CLS_CHUNK, CLS_CHUNK)])
        load_idx(lax.min(i1 + 1, nch - 1), sidx0, didx0)
        pltpu.async_copy(a_hbm.at[sidx0], ra0, sem_a0)
        pltpu.async_copy(b_hbm.at[didx0], rb0, sem_b0)
        pltpu.make_async_copy(a_hbm.at[sidx1], ra1, sem_a1).wait()
        pltpu.make_async_copy(b_hbm.at[didx1], rb1, sem_b1).wait()
        addrows(ra1, rb1)
        pltpu.sync_copy(ra1, h0_out.at[pl.ds(obase + i1 * CLS_CHUNK, CLS_CHUNK)])
        return 0

    lax.fori_loop(0, (nch - 1) // 2, ring, 0)
    pltpu.make_async_copy(a_hbm.at[sidx0], ra0, sem_a0).wait()
    pltpu.make_async_copy(b_hbm.at[didx0], rb0, sem_b0).wait()


_clsg_calls = [functools.partial(
    pl.kernel, functools.partial(_clsg_body, e_half=h), mesh=_mesh,
    out_type=jax.ShapeDtypeStruct((E_EDGES // 2, FEAT), _f32),
    scratch_types=[
        pltpu.VMEM((CLS_CHUNK,), jnp.int32),
        pltpu.VMEM((CLS_CHUNK,), jnp.int32),
        pltpu.VMEM((CLS_CHUNK,), jnp.int32),
        pltpu.VMEM((CLS_CHUNK,), jnp.int32),
        pltpu.VMEM((CLS_CHUNK, FEAT), _f32),
        pltpu.VMEM((CLS_CHUNK, FEAT), _f32),
        pltpu.VMEM((CLS_CHUNK, FEAT), _f32),
        pltpu.VMEM((CLS_CHUNK, FEAT), _f32),
        pltpu.SemaphoreType.DMA,
        pltpu.SemaphoreType.DMA,
        pltpu.SemaphoreType.DMA,
        pltpu.SemaphoreType.DMA,
    ],
)() for h in (0, 1)]


def _dot(a, b):
    return lax.dot_general(a, b, (((1,), (0,)), ((), ())),
                           preferred_element_type=_f32)


def _prep_tc(d0_ref, d1_ref, x_ref, wa_ref, wb_ref, dinv_o, g_o):
    deg = d0_ref[...] + d1_ref[...] + 1.0
    dinv = lax.rsqrt(deg)
    dinv_o[...] = dinv
    xv = x_ref[...]
    g_o[...] = jnp.concatenate(
        [_dot(xv, wa_ref[...]) * dinv, _dot(xv, wb_ref[...]) * dinv], axis=0)


def _norm_relu(acc, g, dinv, bias, bng, bnb):
    z = dinv * (acc + g) + bias
    mu = jnp.mean(z, axis=0, keepdims=True)
    var = jnp.mean((z - mu) ** 2, axis=0, keepdims=True)
    return jnp.maximum(bng * (z - mu) / jnp.sqrt(var + 1e-5) + bnb, 0.0)


def _branch_pair(accs_ref, g_ref, dinv_ref, ba_ref, bb_ref, bng_ref, bnb_ref):
    dinv = dinv_ref[...]
    accs = accs_ref[...]
    gv = g_ref[...]
    ha = _norm_relu(accs[:N_NODES], gv[:N_NODES], dinv, ba_ref[...],
                    bng_ref[...], bnb_ref[...])
    hb = _norm_relu(accs[N_PAD:N_PAD + N_NODES], gv[N_NODES:], dinv,
                    bb_ref[...], bng_ref[...], bnb_ref[...])
    return ha, hb, dinv


def _mid_tc(accs_ref, g_ref, dinv_ref,
            ba_ref, bb_ref, bng_ref, bnb_ref, wna_ref, wnb_ref, g_o):
    ha, hb, dinv = _branch_pair(accs_ref, g_ref, dinv_ref,
                                ba_ref, bb_ref, bng_ref, bnb_ref)
    g_o[...] = jnp.concatenate(
        [_dot(ha, wna_ref[...]) * dinv, _dot(hb, wnb_ref[...]) * dinv], axis=0)


def _fin_tc(accs_ref, g_ref, dinv_ref,
            ba_ref, bb_ref, bng_ref, bnb_ref, w1s_ref, w1d_ref,
            a_o, b_o):
    ha, hb, dinv = _branch_pair(accs_ref, g_ref, dinv_ref,
                                ba_ref, bb_ref, bng_ref, bnb_ref)
    xc = ha + hb
    a_o[...] = _dot(xc, w1s_ref[...])
    b_o[...] = _dot(xc, w1d_ref[...])


def _dotb(a, b):
    return lax.dot_general(a.astype(jnp.bfloat16), b.astype(jnp.bfloat16),
                           (((1,), (0,)), ((), ())),
                           preferred_element_type=_f32)


def _cls_tc(h0_ref, ea_ref, w1e_ref, b1_ref, w2_ref, b2_ref,
            w3_ref, b3_ref, w4_ref, b4_ref, w5_ref, b5_ref, out_ref):
    h = jnp.maximum(h0_ref[...] + _dot(ea_ref[...], w1e_ref[...])
                    + b1_ref[...], 0.0)
    h = jnp.maximum(_dotb(h, w2_ref[...]) + b2_ref[...], 0.0)
    h = jnp.maximum(_dotb(h, w3_ref[...]) + b3_ref[...], 0.0)
    h = jnp.maximum(_dotb(h, w4_ref[...]) + b4_ref[...], 0.0)
    out_ref[...] = _dotb(h, w5_ref[...]) + b5_ref[...]


def kernel(x, edge_index, edge_attr, params):
    p = params
    src = edge_index[0]
    dst = edge_index[1]

    deg_parts = _deg_call(dst)
    d0 = deg_parts[:N_NODES].reshape(N_NODES, 1)
    d1 = deg_parts[N_PAD:N_PAD + N_NODES].reshape(N_NODES, 1)

    dinv, g = pl.pallas_call(
        _prep_tc,
        out_shape=(jax.ShapeDtypeStruct((N_NODES, 1), _f32),
                   jax.ShapeDtypeStruct((2 * N_NODES, FEAT), _f32)),
    )(d0, d1, x, p['gcn1a_W'], p['gcn1b_W'])

    src2 = jnp.concatenate([src, src + jnp.int32(N_NODES)])

    def mid_layer(g, ba, bb, bng, bnb, wna, wnb):
        accs = _agg_call(g, src2, dst)
        return pl.pallas_call(
            _mid_tc,
            out_shape=jax.ShapeDtypeStruct((2 * N_NODES, FEAT), _f32),
        )(accs, g, dinv,
          ba.reshape(1, -1), bb.reshape(1, -1),
          bng.reshape(1, -1), bnb.reshape(1, -1), wna, wnb)

    g = mid_layer(g, p['gcn1a_b'], p['gcn1b_b'],
                  p['bn1_g'], p['bn1_b'], p['gcn2a_W'], p['gcn2b_W'])
    g = mid_layer(g, p['gcn2a_b'], p['gcn2b_b'],
                  p['bn2_g'], p['bn2_b'], p['gcn3a_W'], p['gcn3b_W'])

    accs = _agg_call(g, src2, dst)
    a_t, b_t = pl.pallas_call(
        _fin_tc,
        out_shape=(jax.ShapeDtypeStruct((N_NODES, FEAT), _f32),
                   jax.ShapeDtypeStruct((N_NODES, FEAT), _f32)),
    )(accs, g, dinv,
      p['gcn3a_b'].reshape(1, -1), p['gcn3b_b'].reshape(1, -1),
      p['bn3_g'].reshape(1, -1), p['bn3_b'].reshape(1, -1),
      p['cls_W1'][:FEAT], p['cls_W1'][FEAT:2 * FEAT])

    E2 = E_EDGES // 2
    nblk = E2 // BE

    def cls_half(h0, ea):
        return pl.pallas_call(
            _cls_tc,
            grid=(nblk,),
            in_specs=[
                pl.BlockSpec((BE, FEAT), lambda i: (i, 0)),
                pl.BlockSpec((BE, 16), lambda i: (i, 0)),
                pl.BlockSpec((16, FEAT), lambda i: (0, 0)),
                pl.BlockSpec((1, FEAT), lambda i: (0, 0)),
                pl.BlockSpec((FEAT, FEAT), lambda i: (0, 0)),
                pl.BlockSpec((1, FEAT), lambda i: (0, 0)),
                pl.BlockSpec((FEAT, 64), lambda i: (0, 0)),
                pl.BlockSpec((1, 64), lambda i: (0, 0)),
                pl.BlockSpec((64, 32), lambda i: (0, 0)),
                pl.BlockSpec((1, 32), lambda i: (0, 0)),
                pl.BlockSpec((32, 2), lambda i: (0, 0)),
                pl.BlockSpec((1, 2), lambda i: (0, 0)),
            ],
            out_specs=pl.BlockSpec((BE, 2), lambda i: (i, 0)),
            out_shape=jax.ShapeDtypeStruct((E2, 2), _f32),
        )(h0, ea, p['cls_W1'][2 * FEAT:], p['cls_b1'].reshape(1, -1),
          p['cls_W2'], p['cls_b2'].reshape(1, -1),
          p['cls_W3'], p['cls_b3'].reshape(1, -1),
          p['cls_W4'], p['cls_b4'].reshape(1, -1),
          p['cls_W5'], p['cls_b5'].reshape(1, -1))

    h0_0 = _clsg_calls[0](a_t, b_t, src, dst)
    h0_1 = _clsg_calls[1](a_t, b_t, src, dst)
    out0 = cls_half(h0_0, edge_attr[:E2])
    out1 = cls_half(h0_1, edge_attr[E2:])
    return jnp.concatenate([out0, out1], axis=0)


# speedup vs baseline: 1.1394x; 1.0080x over previous
import functools

import jax
import jax.numpy as jnp
from jax import lax
from jax.experimental import pallas as pl
from jax.experimental.pallas import tpu as pltpu
from jax.experimental.pallas import tpu_sc as plsc

N_NODES = 10000
N_PAD = 10240
E_EDGES = 320000
FEAT = 128
CHUNK = 400
AGG_CHUNK = 160
CLS_CHUNK = 200
BE = 8000

_f32 = jnp.float32
_mesh = plsc.VectorSubcoreMesh(core_axis_name="c", subcore_axis_name="s")


def _fill(ref, rows, width, value):
    vec = jnp.full((16,), value, _f32)
    if len(ref.shape) == 1:
        def body(i, _):
            ref[pl.ds(i * 16, 16)] = vec
            return 0
        lax.fori_loop(0, rows * width // 16, body, 0)
    else:
        def body(i, _):
            for k in range(width // 16):
                ref[i, pl.ds(k * 16, 16)] = vec
            return 0
        lax.fori_loop(0, rows, body, 0)


def _deg_body(dst_hbm, deg_out, idx_v, ones_v, zb_v, acc_sh, sem):
    c = lax.axis_index("c")
    s = lax.axis_index("s")
    _fill(zb_v, 640, 1, 0.0)
    _fill(ones_v, CHUNK, 1, 1.0)
    pltpu.sync_copy(zb_v, acc_sh.at[pl.ds(s * 640, 640)])
    plsc.subcore_barrier()
    per_tile = E_EDGES // 32
    base = (c * 16 + s) * per_tile

    def chunk(i, _):
        pltpu.sync_copy(dst_hbm.at[pl.ds(base + i * CHUNK, CHUNK)], idx_v)
        pltpu.sync_copy(ones_v, acc_sh.at[idx_v], add=True)
        return 0

    lax.fori_loop(0, per_tile // CHUNK, chunk, 0)
    plsc.subcore_barrier()
    pltpu.sync_copy(acc_sh.at[pl.ds(s * 640, 640)],
                    deg_out.at[pl.ds(c * N_PAD + s * 640, 640)])


_deg_call = functools.partial(
    pl.kernel, _deg_body, mesh=_mesh,
    out_type=jax.ShapeDtypeStruct((2 * N_PAD,), _f32),
    scratch_types=[
        pltpu.VMEM((CHUNK,), jnp.int32),
        pltpu.VMEM((CHUNK,), _f32),
        pltpu.VMEM((640,), _f32),
        pltpu.VMEM_SHARED((N_PAD,), _f32),
        pltpu.SemaphoreType.DMA,
    ],
)()


def _agg_body(g_hbm, src2_hbm, dst_hbm, out,
              sidx_a, didx_a, sidx_b, didx_b, rows_a, rows_b,
              acc_sh, sem_a, sem_b):
    c = lax.axis_index("c")
    s = lax.axis_index("s")
    _fill(rows_a, AGG_CHUNK, FEAT, 0.0)
    for j in range(4):
        pltpu.sync_copy(rows_a.at[pl.ds(0, AGG_CHUNK)],
                        acc_sh.at[pl.ds(s * 640 + j * AGG_CHUNK, AGG_CHUNK)])
    plsc.subcore_barrier()
    per_tile = E_EDGES // 16
    nch = per_tile // AGG_CHUNK
    base = c * E_EDGES + s * per_tile
    dbase = s * per_tile

    def load_idx(i, si, di):
        pltpu.sync_copy(src2_hbm.at[pl.ds(base + i * AGG_CHUNK, AGG_CHUNK)], si)
        pltpu.sync_copy(dst_hbm.at[pl.ds(dbase + i * AGG_CHUNK, AGG_CHUNK)], di)

    load_idx(0, sidx_a, didx_a)
    pltpu.async_copy(g_hbm.at[sidx_a], rows_a, sem_a).wait()
    pltpu.sync_copy(rows_a, acc_sh.at[didx_a], add=True)

    load_idx(1, sidx_a, didx_a)
    pltpu.async_copy(g_hbm.at[sidx_a], rows_a, sem_a)

    def ring(j, _):
        i0 = 1 + 2 * j
        i1 = i0 + 1
        load_idx(i1, sidx_b, didx_b)
        pltpu.make_async_copy(g_hbm.at[sidx_a], rows_a, sem_a).wait()
        pltpu.async_copy(g_hbm.at[sidx_b], rows_b, sem_b)
        pltpu.sync_copy(rows_a, acc_sh.at[didx_a], add=True)
        load_idx(lax.min(i1 + 1, nch - 1), sidx_a, didx_a)
        pltpu.make_async_copy(g_hbm.at[sidx_b], rows_b, sem_b).wait()
        pltpu.async_copy(g_hbm.at[sidx_a], rows_a, sem_a)
        pltpu.sync_copy(rows_b, acc_sh.at[didx_b], add=True)
        return 0

    lax.fori_loop(0, (nch - 1) // 2, ring, 0)
    pltpu.make_async_copy(g_hbm.at[sidx_a], rows_a, sem_a).wait()
    plsc.subcore_barrier()
    pltpu.sync_copy(acc_sh.at[pl.ds(s * 640, 640)],
                    out.at[pl.ds(c * N_PAD + s * 640, 640)])


_agg_call = functools.partial(
    pl.kernel, _agg_body, mesh=_mesh,
    out_type=jax.ShapeDtypeStruct((2 * N_PAD, FEAT), _f32),
    scratch_types=[
        pltpu.VMEM((AGG_CHUNK,), jnp.int32),
        pltpu.VMEM((AGG_CHUNK,), jnp.int32),
        pltpu.VMEM((AGG_CHUNK,), jnp.int32),
        pltpu.VMEM((AGG_CHUNK,), jnp.int32),
        pltpu.VMEM((AGG_CHUNK, FEAT), _f32),
        pltpu.VMEM((AGG_CHUNK, FEAT), _f32),
        pltpu.VMEM_SHARED((N_PAD, FEAT), _f32),
        pltpu.SemaphoreType.DMA,
        pltpu.SemaphoreType.DMA,
    ],
)()


def _clsg_body(a_hbm, b_hbm, src_hbm, dst_hbm, h0_out,
               sidx0, didx0, sidx1, didx1, ra0, rb0, ra1, rb1,
               sem_a0, sem_b0, sem_a1, sem_b1, *, e_half):
    c = lax.axis_index("c")
    s = lax.axis_index("s")
    per_tile = E_EDGES // 2 // 32
    base = e_half * (E_EDGES // 2) + (c * 16 + s) * per_tile
    obase = (c * 16 + s) * per_tile
    nch = per_tile // CLS_CHUNK

    def load_idx(i, si, di):
        pltpu.sync_copy(src_hbm.at[pl.ds(base + i * CLS_CHUNK, CLS_CHUNK)], si)
        pltpu.sync_copy(dst_hbm.at[pl.ds(base + i * CLS_CHUNK, CLS_CHUNK)], di)

    def addrows(ra, rb):
        def addrow(r, _):
            for k in range(FEAT // 16):
                sl = pl.ds(k * 16, 16)
                ra[r, sl] = ra[r, sl] + rb[r, sl]
            return 0
        lax.fori_loop(0, CLS_CHUNK, addrow, 0)

    load_idx(0, sidx0, didx0)
    pltpu.async_copy(a_hbm.at[sidx0], ra0, sem_a0)
    pltpu.async_copy(b_hbm.at[didx0], rb0, sem_b0)
    pltpu.make_async_copy(a_hbm.at[sidx0], ra0, sem_a0).wait()
    pltpu.make_async_copy(b_hbm.at[didx0], rb0, sem_b0).wait()
    addrows(ra0, rb0)
    pltpu.sync_copy(ra0, h0_out.at[pl.ds(obase, CLS_CHUNK)])

    load_idx(1, sidx0, didx0)
    pltpu.async_copy(a_hbm.at[sidx0], ra0, sem_a0)
    pltpu.async_copy(b_hbm.at[didx0], rb0, sem_b0)

    def ring(j, _):
        i0 = 1 + 2 * j
        i1 = i0 + 1
        load_idx(i1, sidx1, didx1)
        pltpu.async_copy(a_hbm.at[sidx1], ra1, sem_a1)
        pltpu.async_copy(b_hbm.at[didx1], rb1, sem_b1)
        pltpu.make_async_copy(a_hbm.at[sidx0], ra0, sem_a0).wait()
        pltpu.make_async_copy(b_hbm.at[didx0], rb0, sem_b0).wait()
        addrows(ra0, rb0)
        pltpu.sync_copy(ra0, h0_out.at[pl.ds(obase + i0 * CLS_CHUNK, CLS_CHUNK)])
        load_idx(lax.min(i1 + 1, nch - 1), sidx0, didx0)
        pltpu.async_copy(a_hbm.at[sidx0], ra0, sem_a0)
        pltpu.async_copy(b_hbm.at[didx0], rb0, sem_b0)
        pltpu.make_async_copy(a_hbm.at[sidx1], ra1, sem_a1).wait()
        pltpu.make_async_copy(b_hbm.at[didx1], rb1, sem_b1).wait()
        addrows(ra1, rb1)
        pltpu.sync_copy(ra1, h0_out.at[pl.ds(obase + i1 * CLS_CHUNK, CLS_CHUNK)])
        return 0

    lax.fori_loop(0, (nch - 1) // 2, ring, 0)
    pltpu.make_async_copy(a_hbm.at[sidx0], ra0, sem_a0).wait()
    pltpu.make_async_copy(b_hbm.at[didx0], rb0, sem_b0).wait()


_clsg_calls = [functools.partial(
    pl.kernel, functools.partial(_clsg_body, e_half=h), mesh=_mesh,
    out_type=jax.ShapeDtypeStruct((E_EDGES // 2, FEAT), _f32),
    scratch_types=[
        pltpu.VMEM((CLS_CHUNK,), jnp.int32),
        pltpu.VMEM((CLS_CHUNK,), jnp.int32),
        pltpu.VMEM((CLS_CHUNK,), jnp.int32),
        pltpu.VMEM((CLS_CHUNK,), jnp.int32),
        pltpu.VMEM((CLS_CHUNK, FEAT), _f32),
        pltpu.VMEM((CLS_CHUNK, FEAT), _f32),
        pltpu.VMEM((CLS_CHUNK, FEAT), _f32),
        pltpu.VMEM((CLS_CHUNK, FEAT), _f32),
        pltpu.SemaphoreType.DMA,
        pltpu.SemaphoreType.DMA,
        pltpu.SemaphoreType.DMA,
        pltpu.SemaphoreType.DMA,
    ],
)() for h in (0, 1)]


def _dot(a, b):
    return lax.dot_general(a, b, (((1,), (0,)), ((), ())),
                           preferred_element_type=_f32)


def _prep_tc(d0_ref, d1_ref, x_ref, wa_ref, wb_ref, dinv_o, g_o):
    deg = d0_ref[...] + d1_ref[...] + 1.0
    dinv = lax.rsqrt(deg)
    dinv_o[...] = dinv
    xv = x_ref[...]
    g_o[...] = jnp.concatenate(
        [_dot(xv, wa_ref[...]) * dinv, _dot(xv, wb_ref[...]) * dinv], axis=0)


def _norm_relu(acc, g, dinv, bias, bng, bnb):
    z = dinv * (acc + g) + bias
    mu = jnp.mean(z, axis=0, keepdims=True)
    var = jnp.mean((z - mu) ** 2, axis=0, keepdims=True)
    return jnp.maximum(bng * (z - mu) / jnp.sqrt(var + 1e-5) + bnb, 0.0)


def _branch_pair(accs_ref, g_ref, dinv_ref, ba_ref, bb_ref, bng_ref, bnb_ref):
    dinv = dinv_ref[...]
    accs = accs_ref[...]
    gv = g_ref[...]
    ha = _norm_relu(accs[:N_NODES], gv[:N_NODES], dinv, ba_ref[...],
                    bng_ref[...], bnb_ref[...])
    hb = _norm_relu(accs[N_PAD:N_PAD + N_NODES], gv[N_NODES:], dinv,
                    bb_ref[...], bng_ref[...], bnb_ref[...])
    return ha, hb, dinv


def _mid_tc(accs_ref, g_ref, dinv_ref,
            ba_ref, bb_ref, bng_ref, bnb_ref, wna_ref, wnb_ref, g_o):
    ha, hb, dinv = _branch_pair(accs_ref, g_ref, dinv_ref,
                                ba_ref, bb_ref, bng_ref, bnb_ref)
    g_o[...] = jnp.concatenate(
        [_dot(ha, wna_ref[...]) * dinv, _dot(hb, wnb_ref[...]) * dinv], axis=0)


def _fin_tc(accs_ref, g_ref, dinv_ref,
            ba_ref, bb_ref, bng_ref, bnb_ref, w1s_ref, w1d_ref,
            a_o, b_o):
    ha, hb, dinv = _branch_pair(accs_ref, g_ref, dinv_ref,
                                ba_ref, bb_ref, bng_ref, bnb_ref)
    xc = ha + hb
    a_o[...] = _dot(xc, w1s_ref[...])
    b_o[...] = _dot(xc, w1d_ref[...])


def _dotb(a, b):
    return lax.dot_general(a.astype(jnp.bfloat16), b.astype(jnp.bfloat16),
                           (((1,), (0,)), ((), ())),
                           preferred_element_type=_f32)


def _cls_tc(h0_ref, ea_ref, w1e_ref, b1_ref, w2_ref, b2_ref,
            w3_ref, b3_ref, w4_ref, b4_ref, w5_ref, b5_ref, out_ref):
    h = jnp.maximum(h0_ref[...] + _dot(ea_ref[...], w1e_ref[...])
                    + b1_ref[...], 0.0)
    h = jnp.maximum(_dotb(h, w2_ref[...]) + b2_ref[...], 0.0)
    h = jnp.maximum(_dotb(h, w3_ref[...]) + b3_ref[...], 0.0)
    h = jnp.maximum(_dotb(h, w4_ref[...]) + b4_ref[...], 0.0)
    out_ref[...] = _dotb(h, w5_ref[...]) + b5_ref[...]


def kernel(x, edge_index, edge_attr, params):
    p = params
    src = edge_index[0]
    dst = edge_index[1]

    deg_parts = _deg_call(dst)
    d0 = deg_parts[:N_NODES].reshape(N_NODES, 1)
    d1 = deg_parts[N_PAD:N_PAD + N_NODES].reshape(N_NODES, 1)

    dinv, g = pl.pallas_call(
        _prep_tc,
        out_shape=(jax.ShapeDtypeStruct((N_NODES, 1), _f32),
                   jax.ShapeDtypeStruct((2 * N_NODES, FEAT), _f32)),
    )(d0, d1, x, p['gcn1a_W'], p['gcn1b_W'])

    src2 = jnp.concatenate([src, src + jnp.int32(N_NODES)])

    def mid_layer(g, ba, bb, bng, bnb, wna, wnb):
        accs = _agg_call(g, src2, dst)
        return pl.pallas_call(
            _mid_tc,
            out_shape=jax.ShapeDtypeStruct((2 * N_NODES, FEAT), _f32),
        )(accs, g, dinv,
          ba.reshape(1, -1), bb.reshape(1, -1),
          bng.reshape(1, -1), bnb.reshape(1, -1), wna, wnb)

    g = mid_layer(g, p['gcn1a_b'], p['gcn1b_b'],
                  p['bn1_g'], p['bn1_b'], p['gcn2a_W'], p['gcn2b_W'])
    g = mid_layer(g, p['gcn2a_b'], p['gcn2b_b'],
                  p['bn2_g'], p['bn2_b'], p['gcn3a_W'], p['gcn3b_W'])

    accs = _agg_call(g, src2, dst)
    a_t, b_t = pl.pallas_call(
        _fin_tc,
        out_shape=(jax.ShapeDtypeStruct((N_NODES, FEAT), _f32),
                   jax.ShapeDtypeStruct((N_NODES, FEAT), _f32)),
    )(accs, g, dinv,
      p['gcn3a_b'].reshape(1, -1), p['gcn3b_b'].reshape(1, -1),
      p['bn3_g'].reshape(1, -1), p['bn3_b'].reshape(1, -1),
      p['cls_W1'][:FEAT], p['cls_W1'][FEAT:2 * FEAT])

    E2 = E_EDGES // 2
    nblk = E2 // BE

    def cls_half(h0, ea):
        return pl.pallas_call(
            _cls_tc,
            grid=(nblk,),
            in_specs=[
                pl.BlockSpec((BE, FEAT), lambda i: (i, 0)),
                pl.BlockSpec((BE, 16), lambda i: (i, 0)),
                pl.BlockSpec((16, FEAT), lambda i: (0, 0)),
                pl.BlockSpec((1, FEAT), lambda i: (0, 0)),
                pl.BlockSpec((FEAT, FEAT), lambda i: (0, 0)),
                pl.BlockSpec((1, FEAT), lambda i: (0, 0)),
                pl.BlockSpec((FEAT, 64), lambda i: (0, 0)),
                pl.BlockSpec((1, 64), lambda i: (0, 0)),
                pl.BlockSpec((64, 32), lambda i: (0, 0)),
                pl.BlockSpec((1, 32), lambda i: (0, 0)),
                pl.BlockSpec((32, 2), lambda i: (0, 0)),
                pl.BlockSpec((1, 2), lambda i: (0, 0)),
            ],
            out_specs=pl.BlockSpec((BE, 2), lambda i: (i, 0)),
            out_shape=jax.ShapeDtypeStruct((E2, 2), _f32),
        )(h0, ea, p['cls_W1'][2 * FEAT:], p['cls_b1'].reshape(1, -1),
          p['cls_W2'], p['cls_b2'].reshape(1, -1),
          p['cls_W3'], p['cls_b3'].reshape(1, -1),
          p['cls_W4'], p['cls_b4'].reshape(1, -1),
          p['cls_W5'], p['cls_b5'].reshape(1, -1))

    h0_0 = _clsg_calls[0](a_t, b_t, src, dst)
    h0_1 = _clsg_calls[1](a_t, b_t, src, dst)
    out0 = cls_half(h0_0, edge_attr[:E2])
    out1 = cls_half(h0_1, edge_attr[E2:])
    return jnp.concatenate([out0, out1], axis=0)


# speedup vs baseline: 1.1395x; 1.0000x over previous
import functools

import jax
import jax.numpy as jnp
from jax import lax
from jax.experimental import pallas as pl
from jax.experimental.pallas import tpu as pltpu
from jax.experimental.pallas import tpu_sc as plsc

N_NODES = 10000
N_PAD = 10240
E_EDGES = 320000
FEAT = 128
CHUNK = 400
AGG_CHUNK = 160
CLS_CHUNK = 200
BE = 8000

_f32 = jnp.float32
_mesh = plsc.VectorSubcoreMesh(core_axis_name="c", subcore_axis_name="s")


def _fill(ref, rows, width, value):
    vec = jnp.full((16,), value, _f32)
    if len(ref.shape) == 1:
        def body(i, _):
            ref[pl.ds(i * 16, 16)] = vec
            return 0
        lax.fori_loop(0, rows * width // 16, body, 0)
    else:
        def body(i, _):
            for k in range(width // 16):
                ref[i, pl.ds(k * 16, 16)] = vec
            return 0
        lax.fori_loop(0, rows, body, 0)


def _deg_body(dst_hbm, deg_out, idx_v, ones_v, zb_v, acc_sh, sem):
    c = lax.axis_index("c")
    s = lax.axis_index("s")
    _fill(zb_v, 640, 1, 0.0)
    _fill(ones_v, CHUNK, 1, 1.0)
    pltpu.sync_copy(zb_v, acc_sh.at[pl.ds(s * 640, 640)])
    plsc.subcore_barrier()
    per_tile = E_EDGES // 32
    base = (c * 16 + s) * per_tile

    def chunk(i, _):
        pltpu.sync_copy(dst_hbm.at[pl.ds(base + i * CHUNK, CHUNK)], idx_v)
        pltpu.sync_copy(ones_v, acc_sh.at[idx_v], add=True)
        return 0

    lax.fori_loop(0, per_tile // CHUNK, chunk, 0)
    plsc.subcore_barrier()
    pltpu.sync_copy(acc_sh.at[pl.ds(s * 640, 640)],
                    deg_out.at[pl.ds(c * N_PAD + s * 640, 640)])


_deg_call = functools.partial(
    pl.kernel, _deg_body, mesh=_mesh,
    out_type=jax.ShapeDtypeStruct((2 * N_PAD,), _f32),
    scratch_types=[
        pltpu.VMEM((CHUNK,), jnp.int32),
        pltpu.VMEM((CHUNK,), _f32),
        pltpu.VMEM((640,), _f32),
        pltpu.VMEM_SHARED((N_PAD,), _f32),
        pltpu.SemaphoreType.DMA,
    ],
)()


def _agg_body(g_hbm, src2_hbm, dst_hbm, out,
              sidx_a, didx_a, sidx_b, didx_b, rows_a, rows_b,
              acc_sh, sem_a, sem_b):
    c = lax.axis_index("c")
    s = lax.axis_index("s")
    _fill(rows_a, AGG_CHUNK, FEAT, 0.0)
    for j in range(4):
        pltpu.sync_copy(rows_a.at[pl.ds(0, AGG_CHUNK)],
                        acc_sh.at[pl.ds(s * 640 + j * AGG_CHUNK, AGG_CHUNK)])
    plsc.subcore_barrier()
    per_tile = E_EDGES // 16
    nch = per_tile // AGG_CHUNK
    base = c * E_EDGES + s * per_tile
    dbase = s * per_tile

    def load_idx(i, si, di):
        pltpu.sync_copy(src2_hbm.at[pl.ds(base + i * AGG_CHUNK, AGG_CHUNK)], si)
        pltpu.sync_copy(dst_hbm.at[pl.ds(dbase + i * AGG_CHUNK, AGG_CHUNK)], di)

    load_idx(0, sidx_a, didx_a)
    pltpu.async_copy(g_hbm.at[sidx_a], rows_a, sem_a).wait()
    pltpu.sync_copy(rows_a, acc_sh.at[didx_a], add=True)

    load_idx(1, sidx_a, didx_a)
    pltpu.async_copy(g_hbm.at[sidx_a], rows_a, sem_a)

    def ring(j, _):
        i0 = 1 + 2 * j
        i1 = i0 + 1
        load_idx(i1, sidx_b, didx_b)
        pltpu.make_async_copy(g_hbm.at[sidx_a], rows_a, sem_a).wait()
        pltpu.async_copy(g_hbm.at[sidx_b], rows_b, sem_b)
        pltpu.sync_copy(rows_a, acc_sh.at[didx_a], add=True)
        load_idx(lax.min(i1 + 1, nch - 1), sidx_a, didx_a)
        pltpu.make_async_copy(g_hbm.at[sidx_b], rows_b, sem_b).wait()
        pltpu.async_copy(g_hbm.at[sidx_a], rows_a, sem_a)
        pltpu.sync_copy(rows_b, acc_sh.at[didx_b], add=True)
        return 0

    lax.fori_loop(0, (nch - 1) // 2, ring, 0)
    pltpu.make_async_copy(g_hbm.at[sidx_a], rows_a, sem_a).wait()
    plsc.subcore_barrier()
    pltpu.sync_copy(acc_sh.at[pl.ds(s * 640, 640)],
                    out.at[pl.ds(c * N_PAD + s * 640, 640)])


_agg_call = functools.partial(
    pl.kernel, _agg_body, mesh=_mesh,
    out_type=jax.ShapeDtypeStruct((2 * N_PAD, FEAT), _f32),
    scratch_types=[
        pltpu.VMEM((AGG_CHUNK,), jnp.int32),
        pltpu.VMEM((AGG_CHUNK,), jnp.int32),
        pltpu.VMEM((AGG_CHUNK,), jnp.int32),
        pltpu.VMEM((AGG_CHUNK,), jnp.int32),
        pltpu.VMEM((AGG_CHUNK, FEAT), _f32),
        pltpu.VMEM((AGG_CHUNK, FEAT), _f32),
        pltpu.VMEM_SHARED((N_PAD, FEAT), _f32),
        pltpu.SemaphoreType.DMA,
        pltpu.SemaphoreType.DMA,
    ],
)()


def _clsg_body(a_hbm, b_hbm, src_hbm, dst_hbm, h0_out,
               sidx0, didx0, sidx1, didx1, ra0, rb0, ra1, rb1,
               sem_a0, sem_b0, sem_a1, sem_b1, *, e_half):
    c = lax.axis_index("c")
    s = lax.axis_index("s")
    per_tile = E_EDGES // 2 // 32
    base = e_half * (E_EDGES // 2) + (c * 16 + s) * per_tile
    obase = (c * 16 + s) * per_tile
    nch = per_tile // CLS_CHUNK

    def load_idx(i, si, di):
        pltpu.sync_copy(src_hbm.at[pl.ds(base + i * CLS_CHUNK, CLS_CHUNK)], si)
        pltpu.sync_copy(dst_hbm.at[pl.ds(base + i * CLS_CHUNK, CLS_CHUNK)], di)

    def addrows(ra, rb):
        def addrow(r, _):
            for k in range(FEAT // 16):
                sl = pl.ds(k * 16, 16)
                ra[r, sl] = ra[r, sl] + rb[r, sl]
            return 0
        lax.fori_loop(0, CLS_CHUNK, addrow, 0)

    load_idx(0, sidx0, didx0)
    pltpu.async_copy(a_hbm.at[sidx0], ra0, sem_a0)
    pltpu.async_copy(b_hbm.at[didx0], rb0, sem_b0)
    pltpu.make_async_copy(a_hbm.at[sidx0], ra0, sem_a0).wait()
    pltpu.make_async_copy(b_hbm.at[didx0], rb0, sem_b0).wait()
    addrows(ra0, rb0)
    pltpu.sync_copy(ra0, h0_out.at[pl.ds(obase, CLS_CHUNK)])

    load_idx(1, sidx0, didx0)
    pltpu.async_copy(a_hbm.at[sidx0], ra0, sem_a0)
    pltpu.async_copy(b_hbm.at[didx0], rb0, sem_b0)

    def ring(j, _):
        i0 = 1 + 2 * j
        i1 = i0 + 1
        load_idx(i1, sidx1, didx1)
        pltpu.async_copy(a_hbm.at[sidx1], ra1, sem_a1)
        pltpu.async_copy(b_hbm.at[didx1], rb1, sem_b1)
        pltpu.make_async_copy(a_hbm.at[sidx0], ra0, sem_a0).wait()
        pltpu.make_async_copy(b_hbm.at[didx0], rb0, sem_b0).wait()
        addrows(ra0, rb0)
        pltpu.sync_copy(ra0, h0_out.at[pl.ds(obase + i0 * CLS_CHUNK, CLS_CHUNK)])
        load_idx(lax.min(i1 + 1, nch - 1), sidx0, didx0)
        pltpu.async_copy(a_hbm.at[sidx0], ra0, sem_a0)
        pltpu.async_copy(b_hbm.at[didx0], rb0, sem_b0)
        pltpu.make_async_copy(a_hbm.at[sidx1], ra1, sem_a1).wait()
        pltpu.make_async_copy(b_hbm.at[didx1], rb1, sem_b1).wait()
        addrows(ra1, rb1)
        pltpu.sync_copy(ra1, h0_out.at[pl.ds(obase + i1 * CLS_CHUNK, CLS_CHUNK)])
        return 0

    lax.fori_loop(0, (nch - 1) // 2, ring, 0)
    pltpu.make_async_copy(a_hbm.at[sidx0], ra0, sem_a0).wait()
    pltpu.make_async_copy(b_hbm.at[didx0], rb0, sem_b0).wait()


_clsg_calls = [functools.partial(
    pl.kernel, functools.partial(_clsg_body, e_half=h), mesh=_mesh,
    out_type=jax.ShapeDtypeStruct((E_EDGES // 2, FEAT), _f32),
    scratch_types=[
        pltpu.VMEM((CLS_CHUNK,), jnp.int32),
        pltpu.VMEM((CLS_CHUNK,), jnp.int32),
        pltpu.VMEM((CLS_CHUNK,), jnp.int32),
        pltpu.VMEM((CLS_CHUNK,), jnp.int32),
        pltpu.VMEM((CLS_CHUNK, FEAT), _f32),
        pltpu.VMEM((CLS_CHUNK, FEAT), _f32),
        pltpu.VMEM((CLS_CHUNK, FEAT), _f32),
        pltpu.VMEM((CLS_CHUNK, FEAT), _f32),
        pltpu.SemaphoreType.DMA,
        pltpu.SemaphoreType.DMA,
        pltpu.SemaphoreType.DMA,
        pltpu.SemaphoreType.DMA,
    ],
)() for h in (0, 1)]


def _dot(a, b):
    return lax.dot_general(a, b, (((1,), (0,)), ((), ())),
                           preferred_element_type=_f32)


def _prep_tc(d0_ref, d1_ref, x_ref, wa_ref, wb_ref, dinv_o, g_o):
    deg = d0_ref[...] + d1_ref[...] + 1.0
    dinv = lax.rsqrt(deg)
    dinv_o[...] = dinv
    xv = x_ref[...]
    g_o[...] = jnp.concatenate(
        [_dot(xv, wa_ref[...]) * dinv, _dot(xv, wb_ref[...]) * dinv], axis=0)


def _norm_relu(acc, g, dinv, bias, bng, bnb):
    z = dinv * (acc + g) + bias
    mu = jnp.mean(z, axis=0, keepdims=True)
    var = jnp.mean((z - mu) ** 2, axis=0, keepdims=True)
    return jnp.maximum(bng * (z - mu) / jnp.sqrt(var + 1e-5) + bnb, 0.0)


def _branch_pair(accs_ref, g_ref, dinv_ref, ba_ref, bb_ref, bng_ref, bnb_ref):
    dinv = dinv_ref[...]
    accs = accs_ref[...]
    gv = g_ref[...]
    ha = _norm_relu(accs[:N_NODES], gv[:N_NODES], dinv, ba_ref[...],
                    bng_ref[...], bnb_ref[...])
    hb = _norm_relu(accs[N_PAD:N_PAD + N_NODES], gv[N_NODES:], dinv,
                    bb_ref[...], bng_ref[...], bnb_ref[...])
    return ha, hb, dinv


def _mid_tc(accs_ref, g_ref, dinv_ref,
            ba_ref, bb_ref, bng_ref, bnb_ref, wna_ref, wnb_ref, g_o):
    ha, hb, dinv = _branch_pair(accs_ref, g_ref, dinv_ref,
                                ba_ref, bb_ref, bng_ref, bnb_ref)
    g_o[...] = jnp.concatenate(
        [_dot(ha, wna_ref[...]) * dinv, _dot(hb, wnb_ref[...]) * dinv], axis=0)


def _fin_tc(accs_ref, g_ref, dinv_ref,
            ba_ref, bb_ref, bng_ref, bnb_ref, w1s_ref, w1d_ref,
            a_o, b_o):
    ha, hb, dinv = _branch_pair(accs_ref, g_ref, dinv_ref,
                                ba_ref, bb_ref, bng_ref, bnb_ref)
    xc = ha + hb
    a_o[...] = _dot(xc, w1s_ref[...])
    b_o[...] = _dot(xc, w1d_ref[...])


def _dotb(a, b):
    return lax.dot_general(a.astype(jnp.bfloat16), b.astype(jnp.bfloat16),
                           (((1,), (0,)), ((), ())),
                           preferred_element_type=_f32)


def _cls_tc(h0_ref, ea_ref, w1e_ref, b1_ref, w2_ref, b2_ref,
            w3_ref, b3_ref, w4_ref, b4_ref, w5_ref, b5_ref, out_ref):
    h = jnp.maximum(h0_ref[...] + _dot(ea_ref[...], w1e_ref[...])
                    + b1_ref[...], 0.0)
    h = jnp.maximum(_dot(h, w2_ref[...]) + b2_ref[...], 0.0)
    h = jnp.maximum(_dot(h, w3_ref[...]) + b3_ref[...], 0.0)
    h = jnp.maximum(_dot(h, w4_ref[...]) + b4_ref[...], 0.0)
    out_ref[...] = _dot(h, w5_ref[...]) + b5_ref[...]


def kernel(x, edge_index, edge_attr, params):
    p = params
    src = edge_index[0]
    dst = edge_index[1]

    deg_parts = _deg_call(dst)
    d0 = deg_parts[:N_NODES].reshape(N_NODES, 1)
    d1 = deg_parts[N_PAD:N_PAD + N_NODES].reshape(N_NODES, 1)

    dinv, g = pl.pallas_call(
        _prep_tc,
        out_shape=(jax.ShapeDtypeStruct((N_NODES, 1), _f32),
                   jax.ShapeDtypeStruct((2 * N_NODES, FEAT), _f32)),
    )(d0, d1, x, p['gcn1a_W'], p['gcn1b_W'])

    src2 = jnp.concatenate([src, src + jnp.int32(N_NODES)])

    def mid_layer(g, ba, bb, bng, bnb, wna, wnb):
        accs = _agg_call(g, src2, dst)
        return pl.pallas_call(
            _mid_tc,
            out_shape=jax.ShapeDtypeStruct((2 * N_NODES, FEAT), _f32),
        )(accs, g, dinv,
          ba.reshape(1, -1), bb.reshape(1, -1),
          bng.reshape(1, -1), bnb.reshape(1, -1), wna, wnb)

    g = mid_layer(g, p['gcn1a_b'], p['gcn1b_b'],
                  p['bn1_g'], p['bn1_b'], p['gcn2a_W'], p['gcn2b_W'])
    g = mid_layer(g, p['gcn2a_b'], p['gcn2b_b'],
                  p['bn2_g'], p['bn2_b'], p['gcn3a_W'], p['gcn3b_W'])

    accs = _agg_call(g, src2, dst)
    a_t, b_t = pl.pallas_call(
        _fin_tc,
        out_shape=(jax.ShapeDtypeStruct((N_NODES, FEAT), _f32),
                   jax.ShapeDtypeStruct((N_NODES, FEAT), _f32)),
    )(accs, g, dinv,
      p['gcn3a_b'].reshape(1, -1), p['gcn3b_b'].reshape(1, -1),
      p['bn3_g'].reshape(1, -1), p['bn3_b'].reshape(1, -1),
      p['cls_W1'][:FEAT], p['cls_W1'][FEAT:2 * FEAT])

    E2 = E_EDGES // 2
    nblk = E2 // BE

    def cls_half(h0, ea):
        return pl.pallas_call(
            _cls_tc,
            grid=(nblk,),
            in_specs=[
                pl.BlockSpec((BE, FEAT), lambda i: (i, 0)),
                pl.BlockSpec((BE, 16), lambda i: (i, 0)),
                pl.BlockSpec((16, FEAT), lambda i: (0, 0)),
                pl.BlockSpec((1, FEAT), lambda i: (0, 0)),
                pl.BlockSpec((FEAT, FEAT), lambda i: (0, 0)),
                pl.BlockSpec((1, FEAT), lambda i: (0, 0)),
                pl.BlockSpec((FEAT, 64), lambda i: (0, 0)),
                pl.BlockSpec((1, 64), lambda i: (0, 0)),
                pl.BlockSpec((64, 32), lambda i: (0, 0)),
                pl.BlockSpec((1, 32), lambda i: (0, 0)),
                pl.BlockSpec((32, 2), lambda i: (0, 0)),
                pl.BlockSpec((1, 2), lambda i: (0, 0)),
            ],
            out_specs=pl.BlockSpec((BE, 2), lambda i: (i, 0)),
            out_shape=jax.ShapeDtypeStruct((E2, 2), _f32),
        )(h0, ea, p['cls_W1'][2 * FEAT:], p['cls_b1'].reshape(1, -1),
          p['cls_W2'], p['cls_b2'].reshape(1, -1),
          p['cls_W3'], p['cls_b3'].reshape(1, -1),
          p['cls_W4'], p['cls_b4'].reshape(1, -1),
          p['cls_W5'], p['cls_b5'].reshape(1, -1))

    h0_0 = _clsg_calls[0](a_t, b_t, src, dst)
    h0_1 = _clsg_calls[1](a_t, b_t, src, dst)
    out0 = cls_half(h0_0, edge_attr[:E2])
    out1 = cls_half(h0_1, edge_attr[E2:])
    return jnp.concatenate([out0, out1], axis=0)
